# Initial kernel scaffold; baseline (speedup 1.0000x reference)
import jax
import jax.numpy as jnp
from jax.experimental import pallas as pl


def kernel(x, W, b):
    raise NotImplementedError("write your pallas kernel here")


# speedup vs baseline: 2.2208x; 2.2208x over previous
import functools

import jax
import jax.numpy as jnp
from jax import lax
from jax.experimental import pallas as pl
from jax.experimental.pallas import tpu as pltpu
from jax.experimental.pallas import tpu_sc as plsc

NUM_STATE = 1000
NUM_ACTION = 64
BATCH = 16384

_info = plsc.get_sparse_core_info()
_NC = _info.num_cores
_NS = _info.num_subcores
_L = _info.num_lanes
_NW = _NC * _NS
_BPW = BATCH // _NW
_CHUNK = 128
_NCHUNK = _BPW // _CHUNK
_NBV = NUM_ACTION // _L


@functools.partial(
    pl.kernel,
    out_type=jax.ShapeDtypeStruct((BATCH, NUM_ACTION), jnp.float32),
    mesh=plsc.VectorSubcoreMesh(core_axis_name="c", subcore_axis_name="s"),
    scratch_types=[
        pltpu.VMEM((_NCHUNK, _CHUNK), jnp.int32),
        pltpu.VMEM((_BPW, NUM_ACTION), jnp.float32),
        pltpu.VMEM((NUM_ACTION,), jnp.float32),
        pltpu.SemaphoreType.DMA,
    ],
    compiler_params=pltpu.CompilerParams(use_tc_tiling_on_sc=False),
)
def _qnet_gather(x_hbm, wt_hbm, b_hbm, out_hbm, idx_v, rows_v, b_v, sem):
    wid = lax.axis_index("s") * _NC + lax.axis_index("c")
    base = wid * _BPW

    for j in range(_NCHUNK):
        pltpu.sync_copy(x_hbm.at[pl.ds(base + j * _CHUNK, _CHUNK)], idx_v.at[j])
    pltpu.sync_copy(b_hbm, b_v)

    copies = [
        pltpu.async_copy(
            wt_hbm.at[idx_v.at[j]],
            rows_v.at[pl.ds(j * _CHUNK, _CHUNK)],
            sem,
        )
        for j in range(_NCHUNK)
    ]
    for c in copies:
        c.wait()

    bvals = [b_v[pl.ds(j * _L, _L)] for j in range(_NBV)]

    def _bias(r, carry):
        for j in range(_NBV):
            sl = pl.ds(j * _L, _L)
            rows_v[r, sl] = rows_v[r, sl] + bvals[j]
        return carry

    lax.fori_loop(0, _BPW, _bias, 0)

    pltpu.sync_copy(rows_v, out_hbm.at[pl.ds(base, _BPW)])


def kernel(x, W, b):
    wt = jnp.transpose(W)
    return _qnet_gather(x.astype(jnp.int32), wt, b)
